# Initial kernel scaffold; baseline (speedup 1.0000x reference)
import functools

import jax
import jax.numpy as jnp
from jax.experimental import pallas as pl
from jax.experimental.pallas import tpu as pltpu

_B = 16
_L = 4096
_D = 1024
_BL = 512
_NLB = _L // _BL


def _body(lens_ref, x_ref, o_ref):
    i = pl.program_id(0)
    j = pl.program_id(1)
    ln = lens_ref[i]

    @pl.when(j == 0)
    def _init():
        o_ref[...] = jnp.zeros_like(o_ref)

    nb = jax.lax.div(ln + _BL - 1, _BL)

    @pl.when(j < nb)
    def _acc():
        rel = ln - j * _BL
        rows = jax.lax.broadcasted_iota(jnp.int32, (1, _BL, 1), 1)
        x = jnp.where(rows < rel, x_ref[...], 0.0)
        o_ref[...] += jnp.sum(x, axis=1)

    @pl.when(j == _NLB - 1)
    def _fin():
        o_ref[...] = o_ref[...] / ln.astype(jnp.float32)


def _x_map(i, j, lens):
    nb = jax.lax.div(lens[i] + _BL - 1, _BL)
    jeff = jnp.minimum(j, jnp.maximum(nb - 1, 0))
    return (i, jeff, 0)


@jax.jit
def kernel(V, atoms_lens):
    E = V[0]
    lens = atoms_lens.astype(jnp.int32)
    grid_spec = pltpu.PrefetchScalarGridSpec(
        num_scalar_prefetch=1,
        grid=(_B, _NLB),
        in_specs=[pl.BlockSpec((1, _BL, _D), _x_map)],
        out_specs=pl.BlockSpec((1, _D), lambda i, j, lens: (i, 0)),
    )
    return pl.pallas_call(
        _body,
        grid_spec=grid_spec,
        out_shape=jax.ShapeDtypeStruct((_B, _D), jnp.float32),
    )(lens, E)


# speedup vs baseline: 1.0990x; 1.0990x over previous
import functools

import jax
import jax.numpy as jnp
from jax.experimental import pallas as pl
from jax.experimental.pallas import tpu as pltpu

_B = 16
_L = 4096
_D = 1024
_BL = 512
_NLB = _L // _BL


def _body(lens_ref, x_ref, o_ref):
    i = pl.program_id(0)
    j = pl.program_id(1)
    ln = lens_ref[i]

    @pl.when(j == 0)
    def _init():
        o_ref[...] = jnp.zeros_like(o_ref)

    nb = jax.lax.div(ln + _BL - 1, _BL)

    @pl.when(j < nb)
    def _acc():
        rel = ln - j * _BL
        rows = jax.lax.broadcasted_iota(jnp.int32, (1, _BL, 1), 1)
        x = jnp.where(rows < rel, x_ref[...], 0.0)
        o_ref[...] += jnp.sum(x, axis=1, keepdims=True)

    @pl.when(j == _NLB - 1)
    def _fin():
        o_ref[...] = o_ref[...] / ln.astype(jnp.float32)


def _x_map(i, j, lens):
    nb = jax.lax.div(lens[i] + _BL - 1, _BL)
    jeff = jnp.minimum(j, jnp.maximum(nb - 1, 0))
    return (i, jeff, 0)


@jax.jit
def kernel(V, atoms_lens):
    E = V[0]
    lens = atoms_lens.astype(jnp.int32)
    grid_spec = pltpu.PrefetchScalarGridSpec(
        num_scalar_prefetch=1,
        grid=(_B, _NLB),
        in_specs=[pl.BlockSpec((1, _BL, _D), _x_map)],
        out_specs=pl.BlockSpec((1, 1, _D), lambda i, j, lens: (i, 0, 0)),
    )
    out = pl.pallas_call(
        _body,
        grid_spec=grid_spec,
        out_shape=jax.ShapeDtypeStruct((_B, 1, _D), jnp.float32),
    )(lens, E)
    return out.reshape(_B, _D)
